# Initial kernel scaffold; baseline (speedup 1.0000x reference)
import jax
import jax.numpy as jnp
from jax.experimental import pallas as pl


def kernel(state, action, W1, b1, W2, b2, W3, b3):
    raise NotImplementedError("write your pallas kernel here")


# speedup vs baseline: 1.7751x; 1.7751x over previous
import jax
import jax.numpy as jnp
from jax.experimental import pallas as pl
from jax.experimental.pallas import tpu as pltpu

_S = 64
_A = 16
_H = 256
_E = 7
_TOPK = 5
_D = _S + 1
_IN = _S + _A
_TILE = 512


def _ens_kernel(choice_ref, x_ref, w1_ref, b1_ref, w2_ref, b2_ref, w3_ref,
                b3_ref, eps_ref, ns_ref, rw_ref):
    x = x_ref[...]
    choice = choice_ref[...]
    acc = jnp.zeros((x.shape[0], 2 * 128), jnp.float32)
    for m in range(_TOPK):
        h = jnp.maximum(
            jnp.dot(x, w1_ref[m], preferred_element_type=jnp.float32)
            + b1_ref[m], 0.0)
        h = jnp.maximum(
            jnp.dot(h, w2_ref[m], preferred_element_type=jnp.float32)
            + b2_ref[m], 0.0)
        o = (jnp.dot(h, w3_ref[m], preferred_element_type=jnp.float32)
             + b3_ref[m])
        acc = jnp.where(choice == m, o, acc)
    mu = acc[:, :128]
    log_std = jnp.clip(acc[:, 128:], -20.0, 2.0)
    y = mu + jnp.exp(log_std) * eps_ref[...]
    ns_ref[...] = x[:, :_S] + y[:, :_S]
    rw_ref[...] = y[:, _S:_S + 1]


def kernel(state, action, W1, b1, W2, b2, W3, b3):
    b = state.shape[0]
    x = jnp.concatenate([state, action], axis=-1)

    choice = jax.random.randint(jax.random.key(1), (b,), 0, _TOPK)
    choice = choice.astype(jnp.int32)[:, None]
    eps = jax.random.normal(jax.random.key(2), (b, _D), dtype=state.dtype)
    eps_pad = jnp.zeros((b, 128), state.dtype).at[:, :_D].set(eps)

    w1 = W1[:_TOPK]
    b1p = b1[:_TOPK][:, None, :]
    w2 = W2[:_TOPK]
    b2p = b2[:_TOPK][:, None, :]
    w3p = jnp.zeros((_TOPK, _H, 256), W3.dtype)
    w3p = (w3p.at[:, :, :_D].set(W3[:_TOPK, :, :_D])
               .at[:, :, 128:128 + _D].set(W3[:_TOPK, :, _D:]))
    b3p = jnp.zeros((_TOPK, 1, 256), b3.dtype)
    b3p = (b3p.at[:, 0, :_D].set(b3[:_TOPK, :_D])
               .at[:, 0, 128:128 + _D].set(b3[:_TOPK, _D:]))

    t = _TILE
    grid = (b // t,)
    const3 = lambda i: (0, 0, 0)
    row = lambda i: (i, 0)
    ns, rw = pl.pallas_call(
        _ens_kernel,
        grid=grid,
        in_specs=[
            pl.BlockSpec((t, 1), row),
            pl.BlockSpec((t, _IN), row),
            pl.BlockSpec((_TOPK, _IN, _H), const3),
            pl.BlockSpec((_TOPK, 1, _H), const3),
            pl.BlockSpec((_TOPK, _H, _H), const3),
            pl.BlockSpec((_TOPK, 1, _H), const3),
            pl.BlockSpec((_TOPK, _H, 256), const3),
            pl.BlockSpec((_TOPK, 1, 256), const3),
            pl.BlockSpec((t, 128), row),
        ],
        out_specs=[
            pl.BlockSpec((t, _S), row),
            pl.BlockSpec((t, 1), row),
        ],
        out_shape=[
            jax.ShapeDtypeStruct((b, _S), state.dtype),
            jax.ShapeDtypeStruct((b, 1), state.dtype),
        ],
        compiler_params=pltpu.CompilerParams(
            dimension_semantics=("parallel",)),
    )(choice, x, w1, b1p, w2, b2p, w3p, b3p, eps_pad)
    return (ns, rw)


# speedup vs baseline: 2.1770x; 1.2264x over previous
import functools

import jax
import jax.numpy as jnp
import numpy as np
from jax.experimental import pallas as pl
from jax.experimental.pallas import tpu as pltpu
from jax.experimental.pallas import tpu_sc as plsc

_S = 64
_A = 16
_H = 256
_E = 7
_TOPK = 5
_D = _S + 1
_IN = _S + _A
_TILE = 512
_GW = 128


@functools.lru_cache(maxsize=None)
def _routing(b: int):
    with jax.ensure_compile_time_eval():
        choice = np.asarray(
            jax.random.randint(jax.random.key(1), (b,), 0, _TOPK),
            dtype=np.int64)
        eps = np.asarray(
            jax.random.normal(jax.random.key(2), (b, _D), dtype=jnp.float32))
    perm = np.argsort(choice, kind="stable")
    counts = np.bincount(choice, minlength=_TOPK)
    src_chunks, tile_models = [], []
    dst_pos = np.zeros(b, np.int64)
    off = 0
    pos = 0
    for m in range(_TOPK):
        cnt = int(counts[m])
        rows = perm[off:off + cnt]
        off += cnt
        if cnt == 0:
            continue
        n_t = -(-cnt // _TILE)
        padded = n_t * _TILE
        src_chunks.append(rows)
        src_chunks.append(np.full(padded - cnt, rows[-1], np.int64))
        tile_models += [m] * n_t
        dst_pos[rows] = pos + np.arange(cnt)
        pos += padded
    src = np.concatenate(src_chunks)
    P = int(src.shape[0])
    P1 = -(-P // 4096) * 4096
    src_idx = np.zeros(P1, np.int64)
    src_idx[:P] = src
    eps_pad = np.zeros((b, 128), np.float32)
    eps_pad[:, :_D] = eps
    eps_perm = eps_pad[src_idx[:P]]
    return (src_idx[None, :].astype(np.int32),
            dst_pos[None, :].astype(np.int32),
            np.asarray(tile_models, np.int32), P, P1, eps_perm)


def _sc_mesh():
    return plsc.VectorSubcoreMesh(core_axis_name="c", subcore_axis_name="s")


def _sc_gather_in(x, idx):
    n = idx.shape[1]
    width = x.shape[1]

    @functools.partial(
        pl.kernel, mesh=_sc_mesh(),
        out_type=jax.ShapeDtypeStruct((n, width), x.dtype))
    def gk(x_hbm, i_hbm, o_hbm):
        def body(i_vmem, o_vmem):
            pltpu.sync_copy(x_hbm.at[i_vmem.at[0]], o_vmem)

        pltpu.emit_pipeline(
            body,
            grid=(n // _GW,),
            in_specs=[pl.BlockSpec((1, _GW), lambda i: (0, i))],
            out_specs=[pl.BlockSpec((_GW, width), lambda i: (i, 0))],
            core_axis_name=("c", "s"),
            dimension_semantics=(pltpu.PARALLEL,),
        )(i_hbm, o_hbm)

    return gk(x, idx)


def _mlp_routed(tm_ref, x_ref, eps_ref, w1_ref, b1_ref, w2_ref, b2_ref,
                w3_ref, b3_ref, comb_ref):
    x = x_ref[...]
    h = jnp.maximum(
        jnp.dot(x, w1_ref[0], preferred_element_type=jnp.float32)
        + b1_ref[0], 0.0)
    h = jnp.maximum(
        jnp.dot(h, w2_ref[0], preferred_element_type=jnp.float32)
        + b2_ref[0], 0.0)
    o = (jnp.dot(h, w3_ref[0], preferred_element_type=jnp.float32)
         + b3_ref[0])
    mu = o[:, :128]
    log_std = jnp.clip(o[:, 128:], -20.0, 2.0)
    y = mu + jnp.exp(log_std) * eps_ref[...]
    lane = jax.lax.broadcasted_iota(jnp.int32, x.shape, 1)
    comb_ref[...] = y + jnp.where(lane < _S, x, 0.0)


def kernel(state, action, W1, b1, W2, b2, W3, b3):
    b = state.shape[0]
    src_idx, dst_pos, tile_model, P, P1, eps_perm = _routing(b)
    n_tiles = P // _TILE

    x = jnp.concatenate(
        [state, action, jnp.zeros((b, 128 - _IN), state.dtype)], axis=-1)
    xg = _sc_gather_in(x, jnp.asarray(src_idx))

    w1 = jnp.zeros((_TOPK, 128, _H), W1.dtype).at[:, :_IN, :].set(W1[:_TOPK])
    b1p = b1[:_TOPK][:, None, :]
    w2 = W2[:_TOPK]
    b2p = b2[:_TOPK][:, None, :]
    w3p = jnp.zeros((_TOPK, _H, 256), W3.dtype)
    w3p = (w3p.at[:, :, :_D].set(W3[:_TOPK, :, :_D])
               .at[:, :, 128:128 + _D].set(W3[:_TOPK, :, _D:]))
    b3p = jnp.zeros((_TOPK, 1, 256), b3.dtype)
    b3p = (b3p.at[:, 0, :_D].set(b3[:_TOPK, :_D])
               .at[:, 0, 128:128 + _D].set(b3[:_TOPK, _D:]))

    t = _TILE
    row = lambda i, tm: (i, 0)
    wsel3 = lambda i, tm: (tm[i], 0, 0)
    grid_spec = pltpu.PrefetchScalarGridSpec(
        num_scalar_prefetch=1,
        grid=(n_tiles,),
        in_specs=[
            pl.BlockSpec((t, 128), row),
            pl.BlockSpec((t, 128), row),
            pl.BlockSpec((1, 128, _H), wsel3),
            pl.BlockSpec((1, 1, _H), wsel3),
            pl.BlockSpec((1, _H, _H), wsel3),
            pl.BlockSpec((1, 1, _H), wsel3),
            pl.BlockSpec((1, _H, 256), wsel3),
            pl.BlockSpec((1, 1, 256), wsel3),
        ],
        out_specs=[
            pl.BlockSpec((t, 128), row),
        ],
    )
    [comb] = pl.pallas_call(
        _mlp_routed,
        grid_spec=grid_spec,
        out_shape=[jax.ShapeDtypeStruct((P, 128), state.dtype)],
        compiler_params=pltpu.CompilerParams(
            dimension_semantics=("parallel",)),
    )(jnp.asarray(tile_model), xg, jnp.asarray(eps_perm),
      w1, b1p, w2, b2p, w3p, b3p)

    final = _sc_gather_in(comb, jnp.asarray(dst_pos))
    return (final[:, :_S], final[:, _S:_S + 1])


# speedup vs baseline: 2.2499x; 1.0335x over previous
import functools

import jax
import jax.numpy as jnp
import numpy as np
from jax.experimental import pallas as pl
from jax.experimental.pallas import tpu as pltpu
from jax.experimental.pallas import tpu_sc as plsc

_S = 64
_A = 16
_H = 256
_E = 7
_TOPK = 5
_D = _S + 1
_IN = _S + _A
_TILE = 512
_GW = 128


@functools.lru_cache(maxsize=None)
def _routing(b: int):
    with jax.ensure_compile_time_eval():
        choice = np.asarray(
            jax.random.randint(jax.random.key(1), (b,), 0, _TOPK),
            dtype=np.int64)
        eps = np.asarray(
            jax.random.normal(jax.random.key(2), (b, _D), dtype=jnp.float32))
    perm = np.argsort(choice, kind="stable")
    counts = np.bincount(choice, minlength=_TOPK)
    src_chunks, tile_models = [], []
    dst_pos = np.zeros(b, np.int64)
    off = 0
    pos = 0
    for m in range(_TOPK):
        cnt = int(counts[m])
        rows = perm[off:off + cnt]
        off += cnt
        if cnt == 0:
            continue
        n_t = -(-cnt // _TILE)
        padded = n_t * _TILE
        src_chunks.append(rows)
        src_chunks.append(np.full(padded - cnt, rows[-1], np.int64))
        tile_models += [m] * n_t
        dst_pos[rows] = pos + np.arange(cnt)
        pos += padded
    src = np.concatenate(src_chunks)
    P = int(src.shape[0])
    P1 = -(-P // 4096) * 4096
    src_idx = np.zeros(P1, np.int64)
    src_idx[:P] = src
    eps_pad = np.zeros((b, 128), np.float32)
    eps_pad[:, :_D] = eps
    eps_perm = eps_pad[src_idx[:P]]
    return (src_idx.astype(np.int32), dst_pos.astype(np.int32),
            np.asarray(tile_models, np.int32), P, P1, eps_perm)


def _sc_mesh():
    return plsc.VectorSubcoreMesh(core_axis_name="c", subcore_axis_name="s")


_NBUF = 4
_NWORK = 32


def _sc_gather_rows(src, idx):
    n = idx.shape[0]
    width = src.shape[1]
    nwin_pw = n // (_GW * _NWORK)
    assert n == nwin_pw * _GW * _NWORK

    @functools.partial(
        pl.kernel, mesh=_sc_mesh(),
        out_type=jax.ShapeDtypeStruct((n, width), src.dtype),
        scratch_types=(
            [pltpu.VMEM((nwin_pw * _GW,), jnp.int32)]
            + [pltpu.VMEM((_GW, width), src.dtype) for _ in range(_NBUF)]
            + [pltpu.SemaphoreType.DMA for _ in range(2 * _NBUF)]))
    def gk(src_hbm, i_hbm, o_hbm, idx_v, *bufs_sems):
        bufs = bufs_sems[:_NBUF]
        gsems = bufs_sems[_NBUF:2 * _NBUF]
        ssems = bufs_sems[2 * _NBUF:]
        wid = jax.lax.axis_index("s") * 2 + jax.lax.axis_index("c")
        base = wid * (nwin_pw * _GW)
        pltpu.sync_copy(i_hbm.at[pl.ds(base, nwin_pw * _GW)], idx_v)
        for g in range(0, nwin_pw, _NBUF):
            k = min(_NBUF, nwin_pw - g)
            cps = [
                pltpu.async_copy(
                    src_hbm.at[idx_v.at[pl.ds((g + bi) * _GW, _GW)]],
                    bufs[bi], gsems[bi])
                for bi in range(k)]
            scps = []
            for bi in range(k):
                cps[bi].wait()
                scps.append(pltpu.async_copy(
                    bufs[bi], o_hbm.at[pl.ds(base + (g + bi) * _GW, _GW)],
                    ssems[bi]))
            for scp in scps:
                scp.wait()

    return gk(src, idx)


def _mlp_routed(tm_ref, x_ref, eps_ref, w1_ref, b1_ref, w2_ref, b2_ref,
                w3_ref, b3_ref, comb_ref):
    x = x_ref[...]
    h = jnp.maximum(
        jnp.dot(x, w1_ref[0], preferred_element_type=jnp.float32)
        + b1_ref[0], 0.0)
    h = jnp.maximum(
        jnp.dot(h, w2_ref[0], preferred_element_type=jnp.float32)
        + b2_ref[0], 0.0)
    o = (jnp.dot(h, w3_ref[0], preferred_element_type=jnp.float32)
         + b3_ref[0])
    mu = o[:, :128]
    log_std = jnp.clip(o[:, 128:], -20.0, 2.0)
    y = mu + jnp.exp(log_std) * eps_ref[...]
    lane = jax.lax.broadcasted_iota(jnp.int32, x.shape, 1)
    comb_ref[...] = y + jnp.where(lane < _S, x, 0.0)


def kernel(state, action, W1, b1, W2, b2, W3, b3):
    b = state.shape[0]
    src_idx, dst_pos, tile_model, P, P1, eps_perm = _routing(b)
    n_tiles = P // _TILE

    x = jnp.concatenate(
        [state, action, jnp.zeros((b, 128 - _IN), state.dtype)], axis=-1)
    xg = _sc_gather_rows(x, jnp.asarray(src_idx))

    w1 = jnp.zeros((_TOPK, 128, _H), W1.dtype).at[:, :_IN, :].set(W1[:_TOPK])
    b1p = b1[:_TOPK][:, None, :]
    w2 = W2[:_TOPK]
    b2p = b2[:_TOPK][:, None, :]
    w3p = jnp.zeros((_TOPK, _H, 256), W3.dtype)
    w3p = (w3p.at[:, :, :_D].set(W3[:_TOPK, :, :_D])
               .at[:, :, 128:128 + _D].set(W3[:_TOPK, :, _D:]))
    b3p = jnp.zeros((_TOPK, 1, 256), b3.dtype)
    b3p = (b3p.at[:, 0, :_D].set(b3[:_TOPK, :_D])
               .at[:, 0, 128:128 + _D].set(b3[:_TOPK, _D:]))

    t = _TILE
    row = lambda i, tm: (i, 0)
    wsel3 = lambda i, tm: (tm[i], 0, 0)
    grid_spec = pltpu.PrefetchScalarGridSpec(
        num_scalar_prefetch=1,
        grid=(n_tiles,),
        in_specs=[
            pl.BlockSpec((t, 128), row),
            pl.BlockSpec((t, 128), row),
            pl.BlockSpec((1, 128, _H), wsel3),
            pl.BlockSpec((1, 1, _H), wsel3),
            pl.BlockSpec((1, _H, _H), wsel3),
            pl.BlockSpec((1, 1, _H), wsel3),
            pl.BlockSpec((1, _H, 256), wsel3),
            pl.BlockSpec((1, 1, 256), wsel3),
        ],
        out_specs=[
            pl.BlockSpec((t, 128), row),
        ],
    )
    [comb] = pl.pallas_call(
        _mlp_routed,
        grid_spec=grid_spec,
        out_shape=[jax.ShapeDtypeStruct((P, 128), state.dtype)],
        compiler_params=pltpu.CompilerParams(
            dimension_semantics=("parallel",)),
    )(jnp.asarray(tile_model), xg, jnp.asarray(eps_perm),
      w1, b1p, w2, b2p, w3p, b3p)

    final = _sc_gather_rows(comb, jnp.asarray(dst_pos))
    return (final[:, :_S], final[:, _S:_S + 1])


# speedup vs baseline: 3.3716x; 1.4985x over previous
import functools

import jax
import jax.numpy as jnp
import numpy as np
from jax.experimental import pallas as pl
from jax.experimental.pallas import tpu as pltpu
from jax.experimental.pallas import tpu_sc as plsc

_S = 64
_A = 16
_H = 256
_E = 7
_TOPK = 5
_D = _S + 1
_IN = _S + _A
_TILE = 512
_GW = 128


@functools.lru_cache(maxsize=None)
def _routing(b: int):
    with jax.ensure_compile_time_eval():
        choice = np.asarray(
            jax.random.randint(jax.random.key(1), (b,), 0, _TOPK),
            dtype=np.int64)
        eps = np.asarray(
            jax.random.normal(jax.random.key(2), (b, _D), dtype=jnp.float32))
    perm = np.argsort(choice, kind="stable")
    counts = np.bincount(choice, minlength=_TOPK)
    src_chunks, tile_models = [], []
    dst_pos = np.zeros(b, np.int64)
    off = 0
    pos = 0
    for m in range(_TOPK):
        cnt = int(counts[m])
        rows = perm[off:off + cnt]
        off += cnt
        if cnt == 0:
            continue
        n_t = -(-cnt // _TILE)
        padded = n_t * _TILE
        src_chunks.append(rows)
        src_chunks.append(np.full(padded - cnt, rows[-1], np.int64))
        tile_models += [m] * n_t
        dst_pos[rows] = pos + np.arange(cnt)
        pos += padded
    src = np.concatenate(src_chunks)
    P = int(src.shape[0])
    P1 = -(-P // 4096) * 4096
    src_idx = np.zeros(P1, np.int64)
    src_idx[:P] = src
    eps_pad = np.zeros((b, 128), np.float32)
    eps_pad[:, :_D] = eps
    eps_perm = eps_pad[src_idx[:P]]
    return (src_idx.astype(np.int32), dst_pos.astype(np.int32),
            np.asarray(tile_models, np.int32), P, P1, eps_perm)


def _sc_mesh():
    return plsc.VectorSubcoreMesh(core_axis_name="c", subcore_axis_name="s")


_NBUF = 4
_NWORK = 32


def _sc_scatter_rows(x, idx2d, n_out):
    b = x.shape[0]
    width = x.shape[1]
    nwin_pw = b // (_GW * _NWORK)
    assert b == nwin_pw * _GW * _NWORK

    @functools.partial(
        pl.kernel, mesh=_sc_mesh(),
        out_type=jax.ShapeDtypeStruct((n_out, width), x.dtype),
        scratch_types=(
            [pltpu.VMEM((nwin_pw, _GW), jnp.int32)]
            + [pltpu.VMEM((_GW, width), x.dtype) for _ in range(_NBUF)]
            + [pltpu.SemaphoreType.DMA for _ in range(2 * _NBUF)]))
    def sk(x_hbm, i_hbm, o_hbm, idx_v, *bufs_sems):
        bufs = bufs_sems[:_NBUF]
        rs = bufs_sems[_NBUF:2 * _NBUF]
        ws = bufs_sems[2 * _NBUF:]
        wid = jax.lax.axis_index("s") * 2 + jax.lax.axis_index("c")
        base_w = wid * nwin_pw
        pltpu.sync_copy(i_hbm.at[pl.ds(base_w, nwin_pw)], idx_v)
        for g in range(0, nwin_pw, _NBUF):
            k = min(_NBUF, nwin_pw - g)
            cps = [
                pltpu.async_copy(
                    x_hbm.at[pl.ds((base_w + g + bi) * _GW, _GW)],
                    bufs[bi], rs[bi])
                for bi in range(k)]
            wcps = []
            for bi in range(k):
                cps[bi].wait()
                wcps.append(pltpu.async_copy(
                    bufs[bi], o_hbm.at[idx_v.at[g + bi]], ws[bi]))
            for wcp in wcps:
                wcp.wait()

    return sk(x, idx2d)


def _sc_gather_rows(src, idx):
    n = idx.shape[0]
    width = src.shape[1]
    nwin_pw = n // (_GW * _NWORK)
    assert n == nwin_pw * _GW * _NWORK

    @functools.partial(
        pl.kernel, mesh=_sc_mesh(),
        out_type=jax.ShapeDtypeStruct((n, width), src.dtype),
        scratch_types=(
            [pltpu.VMEM((nwin_pw * _GW,), jnp.int32)]
            + [pltpu.VMEM((_GW, width), src.dtype) for _ in range(_NBUF)]
            + [pltpu.SemaphoreType.DMA for _ in range(2 * _NBUF)]))
    def gk(src_hbm, i_hbm, o_hbm, idx_v, *bufs_sems):
        bufs = bufs_sems[:_NBUF]
        gsems = bufs_sems[_NBUF:2 * _NBUF]
        ssems = bufs_sems[2 * _NBUF:]
        wid = jax.lax.axis_index("s") * 2 + jax.lax.axis_index("c")
        base = wid * (nwin_pw * _GW)
        pltpu.sync_copy(i_hbm.at[pl.ds(base, nwin_pw * _GW)], idx_v)
        for g in range(0, nwin_pw, _NBUF):
            k = min(_NBUF, nwin_pw - g)
            cps = [
                pltpu.async_copy(
                    src_hbm.at[idx_v.at[pl.ds((g + bi) * _GW, _GW)]],
                    bufs[bi], gsems[bi])
                for bi in range(k)]
            scps = []
            for bi in range(k):
                cps[bi].wait()
                scps.append(pltpu.async_copy(
                    bufs[bi], o_hbm.at[pl.ds(base + (g + bi) * _GW, _GW)],
                    ssems[bi]))
            for scp in scps:
                scp.wait()

    return gk(src, idx)


def _mlp_routed(tm_ref, x_ref, eps_ref, w1_ref, b1_ref, w2_ref, b2_ref,
                w3_ref, b3_ref, comb_ref):
    x = x_ref[...]
    h = jnp.maximum(
        jnp.dot(x, w1_ref[0], preferred_element_type=jnp.float32)
        + b1_ref[0], 0.0)
    h = jnp.maximum(
        jnp.dot(h, w2_ref[0], preferred_element_type=jnp.float32)
        + b2_ref[0], 0.0)
    o = (jnp.dot(h, w3_ref[0], preferred_element_type=jnp.float32)
         + b3_ref[0])
    mu = o[:, :128]
    log_std = jnp.clip(o[:, 128:], -20.0, 2.0)
    y = mu + jnp.exp(log_std) * eps_ref[...]
    lane = jax.lax.broadcasted_iota(jnp.int32, x.shape, 1)
    comb_ref[...] = y + jnp.where(lane < _S, x, 0.0)


def kernel(state, action, W1, b1, W2, b2, W3, b3):
    b = state.shape[0]
    src_idx, dst_pos, tile_model, P, P1, eps_perm = _routing(b)
    n_tiles = P // _TILE

    x = jnp.concatenate(
        [state, action, jnp.zeros((b, 128 - _IN), state.dtype)], axis=-1)
    xg = _sc_scatter_rows(x, jnp.asarray(dst_pos.reshape(-1, _GW)), P)

    w1 = jnp.zeros((_TOPK, 128, _H), W1.dtype).at[:, :_IN, :].set(W1[:_TOPK])
    b1p = b1[:_TOPK][:, None, :]
    w2 = W2[:_TOPK]
    b2p = b2[:_TOPK][:, None, :]
    w3p = jnp.zeros((_TOPK, _H, 256), W3.dtype)
    w3p = (w3p.at[:, :, :_D].set(W3[:_TOPK, :, :_D])
               .at[:, :, 128:128 + _D].set(W3[:_TOPK, :, _D:]))
    b3p = jnp.zeros((_TOPK, 1, 256), b3.dtype)
    b3p = (b3p.at[:, 0, :_D].set(b3[:_TOPK, :_D])
               .at[:, 0, 128:128 + _D].set(b3[:_TOPK, _D:]))

    t = _TILE
    row = lambda i, tm: (i, 0)
    wsel3 = lambda i, tm: (tm[i], 0, 0)
    grid_spec = pltpu.PrefetchScalarGridSpec(
        num_scalar_prefetch=1,
        grid=(n_tiles,),
        in_specs=[
            pl.BlockSpec((t, 128), row),
            pl.BlockSpec((t, 128), row),
            pl.BlockSpec((1, 128, _H), wsel3),
            pl.BlockSpec((1, 1, _H), wsel3),
            pl.BlockSpec((1, _H, _H), wsel3),
            pl.BlockSpec((1, 1, _H), wsel3),
            pl.BlockSpec((1, _H, 256), wsel3),
            pl.BlockSpec((1, 1, 256), wsel3),
        ],
        out_specs=[
            pl.BlockSpec((t, 128), row),
        ],
    )
    [comb] = pl.pallas_call(
        _mlp_routed,
        grid_spec=grid_spec,
        out_shape=[jax.ShapeDtypeStruct((P, 128), state.dtype)],
        compiler_params=pltpu.CompilerParams(
            dimension_semantics=("parallel",)),
    )(jnp.asarray(tile_model), xg, jnp.asarray(eps_perm),
      w1, b1p, w2, b2p, w3p, b3p)

    final = _sc_gather_rows(comb, jnp.asarray(dst_pos))
    return (final[:, :_S], final[:, _S:_S + 1])
